# baseline (speedup 1.0000x reference)
import jax
import jax.numpy as jnp
from jax import lax
from jax.experimental import pallas as pl
from jax.experimental.pallas import tpu as pltpu
from jax.experimental.pallas import tpu_sc as plsc

NC, NS = 2, 16
NW = NC * NS
L = 16
S = 500
SPW = 16
SPAD = NW * SPW
SALLOC = SPAD + L
D = 128
DG = D // L
C = 128
NB = 3
NTC = 320
TCW = 408


def _body(x_hbm, sizes_hbm, out_hbm, sizes_v, buf_v, means_v, sem, osem):
    w = lax.axis_index("s") * NC + lax.axis_index("c")
    pltpu.sync_copy(sizes_hbm, sizes_v)

    def t_body(j, tot):
        v = sizes_v[pl.ds(j * L, L)]
        for t in range(L):
            tot = tot + v[t]
        return tot

    t_tc = lax.fori_loop(0, NTC // L, t_body, jnp.int32(0))
    t_sc = lax.fori_loop(NTC // L, SPAD // L, t_body, jnp.int32(0))

    def walk_body(j, carry):
        cum, s_begin, s_end, row_begin, row_end, found = carry
        v = sizes_v[pl.ds(j * L, L)]
        for t in range(L):
            size = v[t]
            s = j * L + t
            m = (2 * (cum - t_tc) + size) * 16
            mine = jnp.logical_and(
                jnp.logical_and(m >= w * t_sc, m < (w + 1) * t_sc),
                size > 0)
            first = jnp.logical_and(mine, found == 0)
            s_begin = jnp.where(first, s, s_begin)
            row_begin = jnp.where(first, cum, row_begin)
            s_end = jnp.where(mine, s + 1, s_end)
            row_end = jnp.where(mine, cum + size, row_end)
            found = jnp.where(mine, jnp.int32(1), found)
            cum = cum + size
        return cum, s_begin, s_end, row_begin, row_end, found

    z = jnp.int32(0)
    _, s_begin, s_end, row_begin, row_end, found = lax.fori_loop(
        NTC // L, SPAD // L, walk_body, (t_tc, z, z, z, z, z))
    s_count = (s_end - s_begin) * found

    def issue(p, base):
        pltpu.async_copy(x_hbm.at[pl.ds(base, C)], buf_v.at[p], sem.at[p])

    def wait(p):
        pltpu.make_async_copy(x_hbm.at[pl.ds(0, C)], buf_v.at[p],
                              sem.at[p]).wait()

    @pl.when(s_count > 0)
    def _process_all():
        alo = (row_begin // 8) * 8
        nch_tot = (row_end - alo + C - 1) // C
        issue(jnp.int32(0), alo)

        @pl.when(nch_tot > 1)
        def _():
            issue(jnp.int32(1), alo + C)

        def seg_body(i, carry):
            start, loaded = carry
            n = sizes_v[pl.ds(s_begin + i, L)][0]
            end = start + n
            c_lo = (start - alo) // C
            c_hi = (end - 1 - alo) // C

            def chunk_body(c, carry):
                acc, loaded = carry
                base = alo + c * C
                p = lax.rem(c, jnp.int32(NB))

                @pl.when(c > loaded)
                def _():
                    wait(p)

                    @pl.when(c + 2 < nch_tot)
                    def _():
                        issue(lax.rem(c + 2, jnp.int32(NB)),
                              alo + (c + 2) * C)

                lo = jnp.maximum(start - base, 0)
                hi = jnp.minimum(end - base, C)

                def row_body(r, a):
                    return tuple(a[f] + buf_v[p, r, pl.ds(f * L, L)]
                                 for f in range(DG))

                return (lax.fori_loop(lo, hi, row_body, acc),
                        jnp.maximum(loaded, c))

            acc0 = tuple(jnp.zeros((L,), jnp.float32) for _ in range(DG))
            acc, loaded = lax.fori_loop(c_lo, c_hi + 1, chunk_body,
                                        (acc0, loaded))

            n_vec = jnp.full((L,), jnp.maximum(n, 1),
                             dtype=jnp.int32).astype(jnp.float32)
            for f in range(DG):
                means_v[i, pl.ds(f * L, L)] = acc[f] / n_vec
            return end, loaded

        lax.fori_loop(0, s_count, seg_body, (row_begin, jnp.int32(-1)))

        ngroups = (s_count + L - 1) // L

        def pad_body(k, o):
            for f in range(DG):
                means_v[k, pl.ds(f * L, L)] = \
                    means_v[s_count - 1, pl.ds(f * L, L)]
            return o

        lax.fori_loop(s_count, ngroups * L, pad_body, z)

        def out_body(g, o):
            idx = jnp.minimum(s_begin + g * L + lax.iota(jnp.int32, 16),
                              s_end - 1)
            src = means_v.at[pl.ds(g * L, L)]
            pltpu.async_copy(src, out_hbm.at[idx], osem)
            pltpu.make_async_copy(src, out_hbm.at[idx], osem).wait()
            return o

        lax.fori_loop(0, ngroups, out_body, z)


_sc_call = pl.kernel(
    _body,
    out_type=jax.ShapeDtypeStruct((SPAD, D), jnp.float32),
    mesh=plsc.VectorSubcoreMesh(core_axis_name="c", subcore_axis_name="s"),
    scratch_types=[
        pltpu.VMEM((SALLOC,), jnp.int32),
        pltpu.VMEM((NB, C, D), jnp.float32),
        pltpu.VMEM((SPAD, D), jnp.float32),
        pltpu.SemaphoreType.DMA((NB,)),
        pltpu.SemaphoreType.DMA,
    ],
)


def _tc_body(starts_ref, sizes_ref, x_hbm, o_ref, buf, sem):
    i = pl.program_id(0)

    def issue(j, p):
        alo = (starts_ref[j] // 8) * 8
        pltpu.make_async_copy(x_hbm.at[pl.ds(alo, TCW)], buf.at[p],
                              sem.at[p]).start()

    @pl.when(i == 0)
    def _():
        issue(0, 0)

    @pl.when(i + 1 < NTC)
    def _():
        issue(i + 1, lax.rem(i + 1, 2))

    p = lax.rem(i, 2)
    pltpu.make_async_copy(x_hbm.at[pl.ds(0, TCW)], buf.at[p],
                          sem.at[p]).wait()
    start = starts_ref[i]
    size = sizes_ref[i]
    off = start - (start // 8) * 8
    rows = buf[p]
    ri = lax.broadcasted_iota(jnp.int32, (TCW, D), 0)
    sel = jnp.logical_and(ri >= off, ri < off + size)
    ssum = jnp.sum(jnp.where(sel, rows, 0.0), axis=0, keepdims=True)
    o_ref[...] = (ssum / size.astype(jnp.float32)).reshape(1, 1, D)


_tc_call = pl.pallas_call(
    _tc_body,
    grid_spec=pltpu.PrefetchScalarGridSpec(
        num_scalar_prefetch=2,
        grid=(NTC,),
        in_specs=[pl.BlockSpec(memory_space=pl.ANY)],
        out_specs=pl.BlockSpec((1, 1, D), lambda i, *_: (i, 0, 0)),
        scratch_shapes=[
            pltpu.VMEM((2, TCW, D), jnp.float32),
            pltpu.SemaphoreType.DMA((2,)),
        ],
    ),
    out_shape=jax.ShapeDtypeStruct((NTC, 1, D), jnp.float32),
    compiler_params=pltpu.CompilerParams(
        dimension_semantics=("arbitrary",)),
)


def kernel(x, feature_size_list):
    sizes = feature_size_list.astype(jnp.int32)
    sizes_pad = jnp.zeros((SALLOC,), jnp.int32).at[:S].set(sizes)
    starts = (jnp.cumsum(sizes) - sizes).astype(jnp.int32)
    tc = _tc_call(starts[:NTC], sizes[:NTC], x).reshape(NTC, D)
    sc = _sc_call(x, sizes_pad)
    return jnp.concatenate([tc, sc[NTC:S]], axis=0)


# speedup vs baseline: 4.1996x; 4.1996x over previous
import jax
import jax.numpy as jnp
from jax import lax
from jax.experimental import pallas as pl
from jax.experimental.pallas import tpu as pltpu
from jax.experimental.pallas import tpu_sc as plsc

NC, NS = 2, 16
NW = NC * NS
L = 16
S = 500
SPW = 16
SPAD = NW * SPW
SALLOC = SPAD + L
D = 128
DG = D // L
C = 112
NB = 4


def _body(x_hbm, sizes_hbm, out_hbm, sizes_v, buf_v, means_v, sem, osem):
    w = lax.axis_index("s") * NC + lax.axis_index("c")
    pltpu.sync_copy(sizes_hbm, sizes_v)

    def t_body(j, tot):
        v = sizes_v[pl.ds(j * L, L)]
        for t in range(L):
            tot = tot + v[t]
        return tot

    total = lax.fori_loop(0, SPAD // L, t_body, jnp.int32(0))

    def walk_body(j, carry):
        cum, s_begin, s_end, row_begin, row_end, found = carry
        v = sizes_v[pl.ds(j * L, L)]
        for t in range(L):
            size = v[t]
            s = j * L + t
            m = (2 * cum + size) * 16
            mine = jnp.logical_and(
                jnp.logical_and(m >= w * total, m < (w + 1) * total),
                size > 0)
            first = jnp.logical_and(mine, found == 0)
            s_begin = jnp.where(first, s, s_begin)
            row_begin = jnp.where(first, cum, row_begin)
            s_end = jnp.where(mine, s + 1, s_end)
            row_end = jnp.where(mine, cum + size, row_end)
            found = jnp.where(mine, jnp.int32(1), found)
            cum = cum + size
        return cum, s_begin, s_end, row_begin, row_end, found

    z = jnp.int32(0)
    _, s_begin, s_end, row_begin, row_end, found = lax.fori_loop(
        0, SPAD // L, walk_body, (z, z, z, z, z, z))
    s_count = (s_end - s_begin) * found

    def issue(p, base):
        pltpu.async_copy(x_hbm.at[pl.ds(base, C)], buf_v.at[p], sem.at[p])

    def wait(p):
        pltpu.make_async_copy(x_hbm.at[pl.ds(0, C)], buf_v.at[p],
                              sem.at[p]).wait()

    @pl.when(s_count > 0)
    def _process_all():
        alo = (row_begin // 8) * 8
        nch_tot = (row_end - alo + C - 1) // C
        issue(jnp.int32(0), alo)

        @pl.when(nch_tot > 1)
        def _():
            issue(jnp.int32(1), alo + C)

        @pl.when(nch_tot > 2)
        def _():
            issue(jnp.int32(2), alo + 2 * C)

        def seg_body(i, carry):
            start, loaded = carry
            n = sizes_v[pl.ds(s_begin + i, L)][0]
            end = start + n
            c_lo = (start - alo) // C
            c_hi = (end - 1 - alo) // C

            def chunk_body(c, carry):
                acc, loaded = carry
                base = alo + c * C
                p = lax.rem(c, jnp.int32(NB))

                @pl.when(c > loaded)
                def _():
                    wait(p)

                    @pl.when(c + 3 < nch_tot)
                    def _():
                        issue(lax.rem(c + 3, jnp.int32(NB)),
                              alo + (c + 3) * C)

                lo = jnp.maximum(start - base, 0)
                hi = jnp.minimum(end - base, C)

                def row_body(r, a):
                    return tuple(a[f] + buf_v[p, r, pl.ds(f * L, L)]
                                 for f in range(DG))

                return (lax.fori_loop(lo, hi, row_body, acc),
                        jnp.maximum(loaded, c))

            acc0 = tuple(jnp.zeros((L,), jnp.float32) for _ in range(DG))
            acc, loaded = lax.fori_loop(c_lo, c_hi + 1, chunk_body,
                                        (acc0, loaded))

            n_vec = jnp.full((L,), jnp.maximum(n, 1),
                             dtype=jnp.int32).astype(jnp.float32)
            for f in range(DG):
                means_v[i, pl.ds(f * L, L)] = acc[f] / n_vec
            return end, loaded

        lax.fori_loop(0, s_count, seg_body, (row_begin, jnp.int32(-1)))

        ngroups = (s_count + L - 1) // L

        def pad_body(k, o):
            for f in range(DG):
                means_v[k, pl.ds(f * L, L)] = \
                    means_v[s_count - 1, pl.ds(f * L, L)]
            return o

        lax.fori_loop(s_count, ngroups * L, pad_body, z)

        def out_body(g, o):
            idx = jnp.minimum(s_begin + g * L + lax.iota(jnp.int32, 16),
                              s_end - 1)
            src = means_v.at[pl.ds(g * L, L)]
            pltpu.async_copy(src, out_hbm.at[idx], osem)
            pltpu.make_async_copy(src, out_hbm.at[idx], osem).wait()
            return o

        lax.fori_loop(0, ngroups, out_body, z)


_sc_call = pl.kernel(
    _body,
    out_type=jax.ShapeDtypeStruct((SPAD, D), jnp.float32),
    mesh=plsc.VectorSubcoreMesh(core_axis_name="c", subcore_axis_name="s"),
    scratch_types=[
        pltpu.VMEM((SALLOC,), jnp.int32),
        pltpu.VMEM((NB, C, D), jnp.float32),
        pltpu.VMEM((SPAD, D), jnp.float32),
        pltpu.SemaphoreType.DMA((NB,)),
        pltpu.SemaphoreType.DMA,
    ],
)


def kernel(x, feature_size_list):
    sizes = jnp.zeros((SALLOC,), jnp.int32).at[:S].set(
        feature_size_list.astype(jnp.int32))
    return _sc_call(x, sizes)[:S]


# speedup vs baseline: 4.2963x; 1.0230x over previous
import jax
import jax.numpy as jnp
from jax import lax
from jax.experimental import pallas as pl
from jax.experimental.pallas import tpu as pltpu
from jax.experimental.pallas import tpu_sc as plsc

NC, NS = 2, 16
NW = NC * NS
L = 16
S = 500
SPW = 16
SPAD = NW * SPW
SALLOC = SPAD + L
D = 128
DG = D // L
C = 112
NB = 4


def _body(x_hbm, sizes_hbm, sizes_t_hbm, out_hbm, sizes_v, sizes_t_v,
          buf_v, means_v, sem, osem):
    w = lax.axis_index("s") * NC + lax.axis_index("c")
    pltpu.sync_copy(sizes_hbm, sizes_v)
    pltpu.sync_copy(sizes_t_hbm, sizes_t_v)

    cs0 = jnp.zeros((L,), jnp.int32)
    cs1 = jnp.zeros((L,), jnp.int32)
    for t in range(L):
        cs0 = cs0 + sizes_t_v[pl.ds(t * 2 * L, L)]
        cs1 = cs1 + sizes_t_v[pl.ds(t * 2 * L + L, L)]

    vt = cs0 + cs1
    total = jnp.int32(0)
    for t in range(L):
        total = total + vt[t]

    z = jnp.int32(0)
    pfx = z
    j_lo, j_hi, cum_lo, seen = z, z, z, z
    for j in range(SPAD // L):
        csj = cs0[j] if j < L else cs1[j - L]
        nxt = pfx + csj
        inter = jnp.logical_and(32 * nxt > w * total,
                                32 * pfx < (w + 1) * total)
        first = jnp.logical_and(inter, seen == 0)
        j_lo = jnp.where(first, j, j_lo)
        cum_lo = jnp.where(first, pfx, cum_lo)
        j_hi = jnp.where(inter, j + 1, j_hi)
        seen = jnp.where(inter, jnp.int32(1), seen)
        pfx = nxt

    def walk_body(j, carry):
        cum, s_begin, s_end, row_begin, row_end, found = carry
        v = sizes_v[pl.ds(j * L, L)]
        for t in range(L):
            size = v[t]
            s = j * L + t
            m = (2 * cum + size) * 16
            mine = jnp.logical_and(
                jnp.logical_and(m >= w * total, m < (w + 1) * total),
                size > 0)
            first = jnp.logical_and(mine, found == 0)
            s_begin = jnp.where(first, s, s_begin)
            row_begin = jnp.where(first, cum, row_begin)
            s_end = jnp.where(mine, s + 1, s_end)
            row_end = jnp.where(mine, cum + size, row_end)
            found = jnp.where(mine, jnp.int32(1), found)
            cum = cum + size
        return cum, s_begin, s_end, row_begin, row_end, found

    _, s_begin, s_end, row_begin, row_end, found = lax.fori_loop(
        j_lo, j_hi, walk_body, (cum_lo, z, z, z, z, z))
    s_count = (s_end - s_begin) * found

    def issue(p, base):
        pltpu.async_copy(x_hbm.at[pl.ds(base, C)], buf_v.at[p], sem.at[p])

    def wait(p):
        pltpu.make_async_copy(x_hbm.at[pl.ds(0, C)], buf_v.at[p],
                              sem.at[p]).wait()

    @pl.when(s_count > 0)
    def _process_all():
        alo = (row_begin // 8) * 8
        nch_tot = (row_end - alo + C - 1) // C
        issue(jnp.int32(0), alo)

        @pl.when(nch_tot > 1)
        def _():
            issue(jnp.int32(1), alo + C)

        @pl.when(nch_tot > 2)
        def _():
            issue(jnp.int32(2), alo + 2 * C)

        def seg_body(i, carry):
            start, loaded = carry
            n = sizes_v[pl.ds(s_begin + i, L)][0]
            end = start + n
            c_lo = (start - alo) // C
            c_hi = (end - 1 - alo) // C

            def chunk_body(c, carry):
                acc, loaded = carry
                base = alo + c * C
                p = lax.rem(c, jnp.int32(NB))

                @pl.when(c > loaded)
                def _():
                    wait(p)

                    @pl.when(c + 3 < nch_tot)
                    def _():
                        issue(lax.rem(c + 3, jnp.int32(NB)),
                              alo + (c + 3) * C)

                lo = jnp.maximum(start - base, 0)
                hi = jnp.minimum(end - base, C)

                def row_body(r, a):
                    return tuple(a[f] + buf_v[p, r, pl.ds(f * L, L)]
                                 for f in range(DG))

                return (lax.fori_loop(lo, hi, row_body, acc),
                        jnp.maximum(loaded, c))

            acc0 = tuple(jnp.zeros((L,), jnp.float32) for _ in range(DG))
            acc, loaded = lax.fori_loop(c_lo, c_hi + 1, chunk_body,
                                        (acc0, loaded))

            n_vec = jnp.full((L,), jnp.maximum(n, 1),
                             dtype=jnp.int32).astype(jnp.float32)
            for f in range(DG):
                means_v[i, pl.ds(f * L, L)] = acc[f] / n_vec
            return end, loaded

        lax.fori_loop(0, s_count, seg_body, (row_begin, jnp.int32(-1)))

        ngroups = (s_count + L - 1) // L

        def pad_body(k, o):
            for f in range(DG):
                means_v[k, pl.ds(f * L, L)] = \
                    means_v[s_count - 1, pl.ds(f * L, L)]
            return o

        lax.fori_loop(s_count, ngroups * L, pad_body, z)

        def out_body(g, o):
            idx = jnp.minimum(s_begin + g * L + lax.iota(jnp.int32, 16),
                              s_end - 1)
            pltpu.async_copy(means_v.at[pl.ds(g * L, L)], out_hbm.at[idx],
                             osem)
            return o

        lax.fori_loop(0, ngroups, out_body, z)

        def drain_body(g, o):
            idx = jnp.minimum(s_begin + g * L + lax.iota(jnp.int32, 16),
                              s_end - 1)
            pltpu.make_async_copy(means_v.at[pl.ds(g * L, L)],
                                  out_hbm.at[idx], osem).wait()
            return o

        lax.fori_loop(0, ngroups, drain_body, z)


_sc_call = pl.kernel(
    _body,
    out_type=jax.ShapeDtypeStruct((SPAD, D), jnp.float32),
    mesh=plsc.VectorSubcoreMesh(core_axis_name="c", subcore_axis_name="s"),
    scratch_types=[
        pltpu.VMEM((SALLOC,), jnp.int32),
        pltpu.VMEM((SPAD,), jnp.int32),
        pltpu.VMEM((NB, C, D), jnp.float32),
        pltpu.VMEM((SPAD, D), jnp.float32),
        pltpu.SemaphoreType.DMA((NB,)),
        pltpu.SemaphoreType.DMA,
    ],
)


def kernel(x, feature_size_list):
    sizes = jnp.zeros((SALLOC,), jnp.int32).at[:S].set(
        feature_size_list.astype(jnp.int32))
    sizes_t = sizes[:SPAD].reshape(SPAD // L, L).T.reshape(SPAD)
    return _sc_call(x, sizes, sizes_t)[:S]
